# baseline (speedup 1.0000x reference)
import functools

import jax
import jax.numpy as jnp
from jax import lax
from jax.experimental import pallas as pl
from jax.experimental.pallas import tpu as pltpu
from jax.experimental.pallas import tpu_sc as plsc

N = 10000
D = 128
NP = 10112
NC, NS = 2, 16
C = 128
E = 320000
CF = 160
SF = 40
EP = NS * CF * C
NCHUNK = CF // 2
RPT = NP // NS
BR = 1264


@functools.lru_cache(maxsize=1)
def _mesh():
    return plsc.VectorSubcoreMesh(
        core_axis_name="c", subcore_axis_name="s",
        num_cores=NC, num_subcores=NS)


def _zero_acc_slice(zbuf, acc, base):
    nfull = RPT // C
    rem = RPT % C

    @pl.loop(0, nfull)
    def _z(k):
        pltpu.sync_copy(zbuf, acc.at[pl.ds(base + k * C, C)])

    if rem:
        pltpu.sync_copy(zbuf.at[pl.ds(0, rem)],
                        acc.at[pl.ds(base + nfull * C, rem)])


def _writeback_slice(acc, out2d, base):
    nfull = RPT // C
    rem = RPT % C

    @pl.loop(0, nfull)
    def _w(k):
        pltpu.sync_copy(acc.at[pl.ds(base + k * C, C)],
                        out2d.at[pl.ds(base + k * C, C)])

    if rem:
        pltpu.sync_copy(acc.at[pl.ds(base + nfull * C, rem)],
                        out2d.at[pl.ds(base + nfull * C, rem)])


def _deg_body(dst_hbm, cnt_hbm, dst_v, ones_v, acc, _sem):
    ci = lax.axis_index("c")
    si = lax.axis_index("s")

    @pl.loop(0, C)
    def _fill0(r):
        for c0 in range(0, D, 16):
            ones_v[pl.ds(r, 1), pl.ds(c0, 16)] = jnp.full((1, 16), 0.0,
                                                          jnp.float32)

    base = si * RPT
    _zero_acc_slice(ones_v, acc, base)

    @pl.loop(0, C)
    def _fill1(r):
        ones_v[pl.ds(r, 1), pl.ds(0, 16)] = jnp.full((1, 16), 1.0, jnp.float32)

    pltpu.sync_copy(dst_hbm.at[ci, si], dst_v)
    plsc.subcore_barrier()

    @pl.loop(0, NCHUNK)
    def _scat(j):
        pltpu.async_copy(ones_v, acc.at[dst_v.at[j]], _sem, add=True)

    @pl.loop(0, NCHUNK)
    def _drain(j):
        pltpu.make_async_copy(ones_v, acc.at[dst_v.at[j]], _sem).wait()

    plsc.subcore_barrier()
    _writeback_slice(acc, cnt_hbm.at[ci], base)


def _deg_call(dstp):
    f = pl.kernel(
        _deg_body,
        out_type=jax.ShapeDtypeStruct((NC, NP, D), jnp.float32),
        mesh=_mesh(),
        scratch_types=[
            pltpu.VMEM((NCHUNK, C), jnp.int32),
            pltpu.VMEM((C, D), jnp.float32),
            pltpu.VMEM_SHARED((NP, D), jnp.float32),
            pltpu.SemaphoreType.DMA,
        ],
    )
    return f(dstp)


def _scat_body(hs_hbm, src1_hbm, dst1_hbm, out_hbm,
               src_v, dst_v, rows0, rows1, acc, sem0, sem1):
    ci = lax.axis_index("c")
    si = lax.axis_index("s")
    base = si * RPT

    @pl.when(ci == 1)
    def _z():
        @pl.loop(0, C)
        def _zrows(r):
            for c0 in range(0, D, 16):
                rows0[pl.ds(r, 1), pl.ds(c0, 16)] = jnp.full((1, 16), 0.0,
                                                             jnp.float32)

        _zero_acc_slice(rows0, acc, base)

    plsc.subcore_barrier()

    def _stage(src_hbm, dst_hbm, t, sf):
        pltpu.sync_copy(src_hbm.at[si, pl.ds(t * sf, sf)],
                        src_v.at[pl.ds(0, sf)])
        pltpu.sync_copy(dst_hbm.at[si, pl.ds(t * sf, sf)],
                        dst_v.at[pl.ds(0, sf)])
        pltpu.async_copy(hs_hbm.at[src_v.at[0]], rows0, sem0)

        @pl.loop(0, sf // 2)
        def _pipe(k):
            j0 = 2 * k
            pltpu.make_async_copy(hs_hbm.at[src_v.at[j0]], rows0, sem0).wait()
            pltpu.async_copy(hs_hbm.at[src_v.at[j0 + 1]], rows1, sem1)
            pltpu.sync_copy(rows0, acc.at[dst_v.at[j0]], add=True)
            pltpu.make_async_copy(hs_hbm.at[src_v.at[j0 + 1]],
                                  rows1, sem1).wait()
            pltpu.async_copy(hs_hbm.at[src_v.at[(j0 + 2) % sf]], rows0, sem0)
            pltpu.sync_copy(rows1, acc.at[dst_v.at[j0 + 1]], add=True)

        pltpu.make_async_copy(hs_hbm.at[src_v.at[0]], rows0, sem0).wait()

    @pl.when(ci == 1)
    def _fast():
        for t in range(CF // SF):
            _stage(src1_hbm, dst1_hbm, t, SF)

    plsc.subcore_barrier()

    @pl.when(ci == 1)
    def _w():
        _writeback_slice(acc, out_hbm, base)


def _scat_call(hs, src1, dst1):
    f = pl.kernel(
        _scat_body,
        out_type=jax.ShapeDtypeStruct((NP, D), jnp.float32),
        mesh=_mesh(),
        scratch_types=[
            pltpu.VMEM((SF, C), jnp.int32),
            pltpu.VMEM((SF, C), jnp.int32),
            pltpu.VMEM((C, D), jnp.float32),
            pltpu.VMEM((C, D), jnp.float32),
            pltpu.VMEM_SHARED((NP, D), jnp.float32),
            pltpu.SemaphoreType.DMA,
            pltpu.SemaphoreType.DMA,
        ],
    )
    return f(hs, src1, dst1)


def _mm_body(x_ref, w_ref, o_ref):
    xb = x_ref[...]
    xb = jnp.where(jnp.isfinite(xb), xb, 0.0)
    o_ref[...] = jnp.dot(xb, w_ref[...], preferred_element_type=jnp.float32)


def _mm_call(xp, wT):
    return pl.pallas_call(
        _mm_body,
        grid=(NP // BR,),
        in_specs=[pl.BlockSpec((BR, D), lambda i: (i, 0)),
                  pl.BlockSpec((D, D), lambda i: (0, 0))],
        out_specs=pl.BlockSpec((BR, D), lambda i: (i, 0)),
        out_shape=jax.ShapeDtypeStruct((NP, D), jnp.float32),
    )(xp, wT)


def _dis(p_ref):
    cnt = p_ref[0, :, 0:1] + p_ref[1, :, 0:1] + 1.0
    return lax.rsqrt(cnt)


def _scale_body(p_ref, h_ref, o_ref):
    o_ref[...] = h_ref[...] * _dis(p_ref)


def _scale_call(cnt, h):
    return pl.pallas_call(
        _scale_body,
        grid=(NP // BR,),
        in_specs=[pl.BlockSpec((NC, BR, D), lambda i: (0, i, 0)),
                  pl.BlockSpec((BR, D), lambda i: (i, 0))],
        out_specs=pl.BlockSpec((BR, D), lambda i: (i, 0)),
        out_shape=jax.ShapeDtypeStruct((NP, D), jnp.float32),
    )(cnt, h)


def _fuse_body(m_ref, hs1_ref, p_ref, w2t_ref, b1_ref, bns_ref, bnb_ref, o_ref):
    dis = _dis(p_ref)
    t = (m_ref[...] + hs1_ref[...]) * dis + b1_ref[...]
    z = jnp.maximum(t * bns_ref[...] + bnb_ref[...], 0.0)
    o_ref[...] = jnp.dot(z, w2t_ref[...],
                         preferred_element_type=jnp.float32) * dis


def _fuse_call(m1, hs1, cnt, w2T, b1r, bns, bnb):
    return pl.pallas_call(
        _fuse_body,
        grid=(NP // BR,),
        in_specs=[pl.BlockSpec((BR, D), lambda i: (i, 0)),
                  pl.BlockSpec((BR, D), lambda i: (i, 0)),
                  pl.BlockSpec((NC, BR, D), lambda i: (0, i, 0)),
                  pl.BlockSpec((D, D), lambda i: (0, 0)),
                  pl.BlockSpec((1, D), lambda i: (0, 0)),
                  pl.BlockSpec((1, D), lambda i: (0, 0)),
                  pl.BlockSpec((1, D), lambda i: (0, 0))],
        out_specs=pl.BlockSpec((BR, D), lambda i: (i, 0)),
        out_shape=jax.ShapeDtypeStruct((NP, D), jnp.float32),
    )(m1, hs1, cnt, w2T, b1r, bns, bnb)


def _final_body(m_ref, hs2_ref, p_ref, b2_ref, o_ref):
    o_ref[...] = (m_ref[...] + hs2_ref[...]) * _dis(p_ref) + b2_ref[...]


def _final_call(m2, hs2, cnt, b2r):
    return pl.pallas_call(
        _final_body,
        grid=(NP // BR,),
        in_specs=[pl.BlockSpec((BR, D), lambda i: (i, 0)),
                  pl.BlockSpec((BR, D), lambda i: (i, 0)),
                  pl.BlockSpec((NC, BR, D), lambda i: (0, i, 0)),
                  pl.BlockSpec((1, D), lambda i: (0, 0))],
        out_specs=pl.BlockSpec((BR, D), lambda i: (i, 0)),
        out_shape=jax.ShapeDtypeStruct((NP, D), jnp.float32),
    )(m2, hs2, cnt, b2r)


def kernel(x, edge_index, W1, b1, W2, b2, bn_gamma, bn_beta, bn_mean, bn_var):
    xp = jnp.pad(x, ((0, NP - N), (0, 0)))
    pad = EP - E
    src_all = jnp.concatenate([edge_index[0], jnp.zeros((pad,), jnp.int32)])
    pad_dst = N + (jnp.arange(pad, dtype=jnp.int32) % (NP - N))
    dst_all = jnp.concatenate([edge_index[1], pad_dst])

    src1 = src_all.reshape(NS, CF, C)
    dst1 = dst_all.reshape(NS, CF, C)
    dst_sym = dst_all.reshape(NC, NS, NCHUNK, C)

    cnt = _deg_call(dst_sym)
    h1 = _mm_call(xp, W1.T)
    hs1 = _scale_call(cnt, h1)
    m1 = _scat_call(hs1, src1, dst1)

    bns = (bn_gamma * lax.rsqrt(bn_var + 1e-5)).reshape(1, D)
    bnb = (bn_beta - bn_mean * bns[0]).reshape(1, D)
    hs2 = _fuse_call(m1, hs1, cnt, W2.T, b1.reshape(1, D), bns, bnb)
    m2 = _scat_call(hs2, src1, dst1)
    out = _final_call(m2, hs2, cnt, b2.reshape(1, D))
    return out[:N]


# speedup vs baseline: 1.0408x; 1.0408x over previous
import functools

import jax
import jax.numpy as jnp
from jax import lax
from jax.experimental import pallas as pl
from jax.experimental.pallas import tpu as pltpu
from jax.experimental.pallas import tpu_sc as plsc

N = 10000
D = 128
NP = 10112
NC, NS = 2, 16
C = 128
E = 320000
CF = 104
CS = 56
EP = NS * (CF + CS) * C
E1 = NS * CF * C
NCHUNK = (CF + CS) // 2
RPT = NP // NS
BR = 1264


@functools.lru_cache(maxsize=1)
def _mesh():
    return plsc.VectorSubcoreMesh(
        core_axis_name="c", subcore_axis_name="s",
        num_cores=NC, num_subcores=NS)


def _zero_acc_slice(zbuf, acc, base):
    nfull = RPT // C
    rem = RPT % C

    @pl.loop(0, nfull)
    def _z(k):
        pltpu.sync_copy(zbuf, acc.at[pl.ds(base + k * C, C)])

    if rem:
        pltpu.sync_copy(zbuf.at[pl.ds(0, rem)],
                        acc.at[pl.ds(base + nfull * C, rem)])


def _writeback_slice(acc, out2d, base):
    nfull = RPT // C
    rem = RPT % C

    @pl.loop(0, nfull)
    def _w(k):
        pltpu.sync_copy(acc.at[pl.ds(base + k * C, C)],
                        out2d.at[pl.ds(base + k * C, C)])

    if rem:
        pltpu.sync_copy(acc.at[pl.ds(base + nfull * C, rem)],
                        out2d.at[pl.ds(base + nfull * C, rem)])


def _deg_body(dst_hbm, cnt_hbm, dst_v, ones_v, acc, _sem):
    ci = lax.axis_index("c")
    si = lax.axis_index("s")

    @pl.loop(0, C)
    def _fill0(r):
        for c0 in range(0, D, 16):
            ones_v[pl.ds(r, 1), pl.ds(c0, 16)] = jnp.full((1, 16), 0.0,
                                                          jnp.float32)

    base = si * RPT
    _zero_acc_slice(ones_v, acc, base)

    @pl.loop(0, C)
    def _fill1(r):
        ones_v[pl.ds(r, 1), pl.ds(0, 16)] = jnp.full((1, 16), 1.0, jnp.float32)

    pltpu.sync_copy(dst_hbm.at[ci, si], dst_v)
    plsc.subcore_barrier()

    @pl.loop(0, NCHUNK)
    def _scat(j):
        pltpu.async_copy(ones_v, acc.at[dst_v.at[j]], _sem, add=True)

    @pl.loop(0, NCHUNK)
    def _drain(j):
        pltpu.make_async_copy(ones_v, acc.at[dst_v.at[j]], _sem).wait()

    plsc.subcore_barrier()
    _writeback_slice(acc, cnt_hbm.at[ci], base)


def _deg_call(dstp):
    f = pl.kernel(
        _deg_body,
        out_type=jax.ShapeDtypeStruct((NC, NP, D), jnp.float32),
        mesh=_mesh(),
        scratch_types=[
            pltpu.VMEM((NCHUNK, C), jnp.int32),
            pltpu.VMEM((C, D), jnp.float32),
            pltpu.VMEM_SHARED((NP, D), jnp.float32),
            pltpu.SemaphoreType.DMA,
        ],
    )
    return f(dstp)


def _scat_body(hs_hbm, src1_hbm, dst1_hbm, src0_hbm, dst0_hbm, out_hbm,
               src_v, dst_v, rows0, acc, sem):
    ci = lax.axis_index("c")
    si = lax.axis_index("s")
    base = si * RPT

    @pl.loop(0, C)
    def _zrows(r):
        for c0 in range(0, D, 16):
            rows0[pl.ds(r, 1), pl.ds(c0, 16)] = jnp.full((1, 16), 0.0,
                                                         jnp.float32)

    _zero_acc_slice(rows0, acc, base)
    plsc.subcore_barrier()

    def _run(src_hbm, dst_hbm, nch):
        pltpu.sync_copy(src_hbm.at[si], src_v.at[pl.ds(0, nch)])
        pltpu.sync_copy(dst_hbm.at[si], dst_v.at[pl.ds(0, nch)])

        @pl.loop(0, nch)
        def _go(j):
            pltpu.async_copy(hs_hbm.at[src_v.at[j]], rows0, sem).wait()
            pltpu.sync_copy(rows0, acc.at[dst_v.at[j]], add=True)

    @pl.when(ci == 1)
    def _fast():
        _run(src1_hbm, dst1_hbm, CF)

    @pl.when(ci == 0)
    def _slow():
        _run(src0_hbm, dst0_hbm, CS)

    plsc.subcore_barrier()
    _writeback_slice(acc, out_hbm.at[ci], base)


def _scat_call(hs, src1, dst1, src0, dst0):
    f = pl.kernel(
        _scat_body,
        out_type=jax.ShapeDtypeStruct((NC, NP, D), jnp.float32),
        mesh=_mesh(),
        scratch_types=[
            pltpu.VMEM((CF, C), jnp.int32),
            pltpu.VMEM((CF, C), jnp.int32),
            pltpu.VMEM((C, D), jnp.float32),
            pltpu.VMEM_SHARED((NP, D), jnp.float32),
            pltpu.SemaphoreType.DMA,
        ],
    )
    return f(hs, src1, dst1, src0, dst0)


def _mm_body(x_ref, w_ref, o_ref):
    xb = x_ref[...]
    xb = jnp.where(jnp.isfinite(xb), xb, 0.0)
    o_ref[...] = jnp.dot(xb, w_ref[...], preferred_element_type=jnp.float32)


def _mm_call(xp, wT):
    return pl.pallas_call(
        _mm_body,
        grid=(NP // BR,),
        in_specs=[pl.BlockSpec((BR, D), lambda i: (i, 0)),
                  pl.BlockSpec((D, D), lambda i: (0, 0))],
        out_specs=pl.BlockSpec((BR, D), lambda i: (i, 0)),
        out_shape=jax.ShapeDtypeStruct((NP, D), jnp.float32),
    )(xp, wT)


def _dis(p_ref):
    cnt = p_ref[0, :, 0:1] + p_ref[1, :, 0:1] + 1.0
    return lax.rsqrt(cnt)


def _scale_body(p_ref, h_ref, o_ref):
    o_ref[...] = h_ref[...] * _dis(p_ref)


def _scale_call(cnt, h):
    return pl.pallas_call(
        _scale_body,
        grid=(NP // BR,),
        in_specs=[pl.BlockSpec((NC, BR, D), lambda i: (0, i, 0)),
                  pl.BlockSpec((BR, D), lambda i: (i, 0))],
        out_specs=pl.BlockSpec((BR, D), lambda i: (i, 0)),
        out_shape=jax.ShapeDtypeStruct((NP, D), jnp.float32),
    )(cnt, h)


def _fuse_body(m_ref, hs1_ref, p_ref, w2t_ref, b1_ref, bns_ref, bnb_ref, o_ref):
    dis = _dis(p_ref)
    t = (m_ref[0] + m_ref[1] + hs1_ref[...]) * dis + b1_ref[...]
    z = jnp.maximum(t * bns_ref[...] + bnb_ref[...], 0.0)
    o_ref[...] = jnp.dot(z, w2t_ref[...],
                         preferred_element_type=jnp.float32) * dis


def _fuse_call(m1, hs1, cnt, w2T, b1r, bns, bnb):
    return pl.pallas_call(
        _fuse_body,
        grid=(NP // BR,),
        in_specs=[pl.BlockSpec((NC, BR, D), lambda i: (0, i, 0)),
                  pl.BlockSpec((BR, D), lambda i: (i, 0)),
                  pl.BlockSpec((NC, BR, D), lambda i: (0, i, 0)),
                  pl.BlockSpec((D, D), lambda i: (0, 0)),
                  pl.BlockSpec((1, D), lambda i: (0, 0)),
                  pl.BlockSpec((1, D), lambda i: (0, 0)),
                  pl.BlockSpec((1, D), lambda i: (0, 0))],
        out_specs=pl.BlockSpec((BR, D), lambda i: (i, 0)),
        out_shape=jax.ShapeDtypeStruct((NP, D), jnp.float32),
    )(m1, hs1, cnt, w2T, b1r, bns, bnb)


def _final_body(m_ref, hs2_ref, p_ref, b2_ref, o_ref):
    o_ref[...] = (m_ref[0] + m_ref[1] + hs2_ref[...]) * _dis(p_ref) \
        + b2_ref[...]


def _final_call(m2, hs2, cnt, b2r):
    return pl.pallas_call(
        _final_body,
        grid=(NP // BR,),
        in_specs=[pl.BlockSpec((NC, BR, D), lambda i: (0, i, 0)),
                  pl.BlockSpec((BR, D), lambda i: (i, 0)),
                  pl.BlockSpec((NC, BR, D), lambda i: (0, i, 0)),
                  pl.BlockSpec((1, D), lambda i: (0, 0))],
        out_specs=pl.BlockSpec((BR, D), lambda i: (i, 0)),
        out_shape=jax.ShapeDtypeStruct((NP, D), jnp.float32),
    )(m2, hs2, cnt, b2r)


def kernel(x, edge_index, W1, b1, W2, b2, bn_gamma, bn_beta, bn_mean, bn_var):
    xp = jnp.pad(x, ((0, NP - N), (0, 0)))
    pad = EP - E
    src_all = jnp.concatenate([edge_index[0], jnp.zeros((pad,), jnp.int32)])
    pad_dst = N + (jnp.arange(pad, dtype=jnp.int32) % (NP - N))
    dst_all = jnp.concatenate([edge_index[1], pad_dst])

    src1 = src_all[:E1].reshape(NS, CF, C)
    dst1 = dst_all[:E1].reshape(NS, CF, C)
    src0 = src_all[E1:].reshape(NS, CS, C)
    dst0 = dst_all[E1:].reshape(NS, CS, C)
    dst_sym = dst_all.reshape(NC, NS, NCHUNK, C)

    cnt = _deg_call(dst_sym)
    h1 = _mm_call(xp, W1.T)
    hs1 = _scale_call(cnt, h1)
    m1 = _scat_call(hs1, src1, dst1, src0, dst0)

    bns = (bn_gamma * lax.rsqrt(bn_var + 1e-5)).reshape(1, D)
    bnb = (bn_beta - bn_mean * bns[0]).reshape(1, D)
    hs2 = _fuse_call(m1, hs1, cnt, W2.T, b1.reshape(1, D), bns, bnb)
    m2 = _scat_call(hs2, src1, dst1, src0, dst0)
    out = _final_call(m2, hs2, cnt, b2.reshape(1, D))
    return out[:N]


# speedup vs baseline: 1.4945x; 1.4359x over previous
import functools

import jax
import jax.numpy as jnp
from jax import lax
from jax.experimental import pallas as pl
from jax.experimental.pallas import tpu as pltpu
from jax.experimental.pallas import tpu_sc as plsc

N = 10000
D = 128
NP = 10240
NC, NS = 2, 16
C = 128
E = 320000
NCHUNK = 79
EP = NC * NS * NCHUNK * C
RPT = NP // NS
BR = 1280


@functools.lru_cache(maxsize=1)
def _mesh():
    return plsc.VectorSubcoreMesh(
        core_axis_name="c", subcore_axis_name="s",
        num_cores=NC, num_subcores=NS)


def _deg_body(dst_hbm, cnt_hbm, dst_v, ones_v, acc, _sem):
    ci = lax.axis_index("c")
    si = lax.axis_index("s")

    @pl.loop(0, C)
    def _fill0(r):
        for c0 in range(0, D, 16):
            ones_v[pl.ds(r, 1), pl.ds(c0, 16)] = jnp.full((1, 16), 0.0,
                                                          jnp.float32)

    base = si * RPT

    @pl.loop(0, RPT // C)
    def _zero(k):
        pltpu.sync_copy(ones_v, acc.at[pl.ds(base + k * C, C)])

    @pl.loop(0, C)
    def _fill1(r):
        ones_v[pl.ds(r, 1), pl.ds(0, 16)] = jnp.full((1, 16), 1.0, jnp.float32)

    pltpu.sync_copy(dst_hbm.at[ci, si], dst_v)
    plsc.subcore_barrier()

    @pl.loop(0, NCHUNK)
    def _scat(j):
        pltpu.sync_copy(ones_v, acc.at[dst_v.at[j]], add=True)

    plsc.subcore_barrier()

    @pl.loop(0, RPT // C)
    def _wb(k):
        pltpu.sync_copy(acc.at[pl.ds(base + k * C, C)],
                        cnt_hbm.at[ci, pl.ds(base + k * C, C)])


def _deg_call(dstp):
    f = pl.kernel(
        _deg_body,
        out_type=jax.ShapeDtypeStruct((NC, NP, D), jnp.float32),
        mesh=_mesh(),
        scratch_types=[
            pltpu.VMEM((NCHUNK, C), jnp.int32),
            pltpu.VMEM((C, D), jnp.float32),
            pltpu.VMEM_SHARED((NP, D), jnp.float32),
            pltpu.SemaphoreType.DMA,
        ],
    )
    return f(dstp)


def _scat_body(hs_hbm, src_hbm, dst_hbm, out_hbm, src_v, dst_v, rows_v, acc,
               sem):
    ci = lax.axis_index("c")
    si = lax.axis_index("s")

    @pl.loop(0, C)
    def _zrows(r):
        for c0 in range(0, D, 16):
            rows_v[pl.ds(r, 1), pl.ds(c0, 16)] = jnp.full((1, 16), 0.0,
                                                          jnp.float32)

    base = si * RPT

    @pl.loop(0, RPT // C)
    def _zero(k):
        pltpu.sync_copy(rows_v, acc.at[pl.ds(base + k * C, C)])

    pltpu.sync_copy(src_hbm.at[ci, si], src_v)
    pltpu.sync_copy(dst_hbm.at[ci, si], dst_v)
    plsc.subcore_barrier()

    @pl.loop(0, NCHUNK)
    def _scat(j):
        pltpu.async_copy(hs_hbm.at[src_v.at[j]], rows_v, sem).wait()
        pltpu.sync_copy(rows_v, acc.at[dst_v.at[j]], add=True)

    plsc.subcore_barrier()

    @pl.loop(0, RPT // C)
    def _wb(k):
        pltpu.sync_copy(acc.at[pl.ds(base + k * C, C)],
                        out_hbm.at[ci, pl.ds(base + k * C, C)])


def _scat_call(hs, srcp, dstp):
    f = pl.kernel(
        _scat_body,
        out_type=jax.ShapeDtypeStruct((NC, NP, D), jnp.float32),
        mesh=_mesh(),
        scratch_types=[
            pltpu.VMEM((NCHUNK, C), jnp.int32),
            pltpu.VMEM((NCHUNK, C), jnp.int32),
            pltpu.VMEM((C, D), jnp.float32),
            pltpu.VMEM_SHARED((NP, D), jnp.float32),
            pltpu.SemaphoreType.DMA,
        ],
    )
    return f(hs, srcp, dstp)


def _mm_body(x_ref, w_ref, o_ref):
    xb = x_ref[...]
    xb = jnp.where(jnp.isfinite(xb), xb, 0.0)
    o_ref[...] = jnp.dot(xb, w_ref[...], preferred_element_type=jnp.float32)


def _mm_call(xp, wT):
    return pl.pallas_call(
        _mm_body,
        grid=(NP // BR,),
        in_specs=[pl.BlockSpec((BR, D), lambda i: (i, 0)),
                  pl.BlockSpec((D, D), lambda i: (0, 0))],
        out_specs=pl.BlockSpec((BR, D), lambda i: (i, 0)),
        out_shape=jax.ShapeDtypeStruct((NP, D), jnp.float32),
    )(xp, wT)


def _dis(p_ref):
    cnt = p_ref[0, :, 0:1] + p_ref[1, :, 0:1] + 1.0
    return lax.rsqrt(cnt)


def _scale_body(p_ref, h_ref, o_ref):
    o_ref[...] = h_ref[...] * _dis(p_ref)


def _scale_call(cnt, h):
    return pl.pallas_call(
        _scale_body,
        grid=(NP // BR,),
        in_specs=[pl.BlockSpec((NC, BR, D), lambda i: (0, i, 0)),
                  pl.BlockSpec((BR, D), lambda i: (i, 0))],
        out_specs=pl.BlockSpec((BR, D), lambda i: (i, 0)),
        out_shape=jax.ShapeDtypeStruct((NP, D), jnp.float32),
    )(cnt, h)


def _fuse_body(m_ref, hs1_ref, p_ref, w2t_ref, b1_ref, bns_ref, bnb_ref, o_ref):
    dis = _dis(p_ref)
    t = (m_ref[0] + m_ref[1] + hs1_ref[...]) * dis + b1_ref[...]
    z = jnp.maximum(t * bns_ref[...] + bnb_ref[...], 0.0)
    o_ref[...] = jnp.dot(z, w2t_ref[...],
                         preferred_element_type=jnp.float32) * dis


def _fuse_call(m1, hs1, cnt, w2T, b1r, bns, bnb):
    return pl.pallas_call(
        _fuse_body,
        grid=(NP // BR,),
        in_specs=[pl.BlockSpec((NC, BR, D), lambda i: (0, i, 0)),
                  pl.BlockSpec((BR, D), lambda i: (i, 0)),
                  pl.BlockSpec((NC, BR, D), lambda i: (0, i, 0)),
                  pl.BlockSpec((D, D), lambda i: (0, 0)),
                  pl.BlockSpec((1, D), lambda i: (0, 0)),
                  pl.BlockSpec((1, D), lambda i: (0, 0)),
                  pl.BlockSpec((1, D), lambda i: (0, 0))],
        out_specs=pl.BlockSpec((BR, D), lambda i: (i, 0)),
        out_shape=jax.ShapeDtypeStruct((NP, D), jnp.float32),
    )(m1, hs1, cnt, w2T, b1r, bns, bnb)


def _final_body(m_ref, hs2_ref, p_ref, b2_ref, o_ref):
    o_ref[...] = (m_ref[0] + m_ref[1] + hs2_ref[...]) * _dis(p_ref) \
        + b2_ref[...]


def _final_call(m2, hs2, cnt, b2r):
    return pl.pallas_call(
        _final_body,
        grid=(NP // BR,),
        in_specs=[pl.BlockSpec((NC, BR, D), lambda i: (0, i, 0)),
                  pl.BlockSpec((BR, D), lambda i: (i, 0)),
                  pl.BlockSpec((NC, BR, D), lambda i: (0, i, 0)),
                  pl.BlockSpec((1, D), lambda i: (0, 0))],
        out_specs=pl.BlockSpec((BR, D), lambda i: (i, 0)),
        out_shape=jax.ShapeDtypeStruct((NP, D), jnp.float32),
    )(m2, hs2, cnt, b2r)


def kernel(x, edge_index, W1, b1, W2, b2, bn_gamma, bn_beta, bn_mean, bn_var):
    xp = jnp.pad(x, ((0, NP - N), (0, 0)))
    pad = EP - E
    srcp = jnp.concatenate(
        [edge_index[0], jnp.zeros((pad,), jnp.int32)]).reshape(NC, NS, NCHUNK, C)
    dstp = jnp.concatenate(
        [edge_index[1], jnp.full((pad,), N, jnp.int32)]).reshape(NC, NS, NCHUNK, C)

    cnt = _deg_call(dstp)
    h1 = _mm_call(xp, W1.T)
    hs1 = _scale_call(cnt, h1)
    m1 = _scat_call(hs1, srcp, dstp)

    bns = (bn_gamma * lax.rsqrt(bn_var + 1e-5)).reshape(1, D)
    bnb = (bn_beta - bn_mean * bns[0]).reshape(1, D)
    hs2 = _fuse_call(m1, hs1, cnt, W2.T, b1.reshape(1, D), bns, bnb)
    m2 = _scat_call(hs2, srcp, dstp)
    out = _final_call(m2, hs2, cnt, b2.reshape(1, D))
    return out[:N]
